# Initial kernel scaffold; baseline (speedup 1.0000x reference)
import jax
import jax.numpy as jnp
from jax.experimental import pallas as pl


def kernel(flat, cu_seqlens):
    raise NotImplementedError("write your pallas kernel here")


# speedup vs baseline: 1.6283x; 1.6283x over previous
import functools

import jax
import jax.numpy as jnp
from jax import lax
from jax.experimental import pallas as pl
from jax.experimental.pallas import tpu as pltpu
from jax.experimental.pallas import tpu_sc as plsc

B = 16
P = 4096
D = 192
T = 32768
NC = 2
NS = 16
NW = NC * NS
HALF = P // 2
CH = 256
LANE = 16

_mesh = plsc.VectorSubcoreMesh(core_axis_name="c", subcore_axis_name="s")


def _body(flat, starts_hbm, ends_hbm, out, mask_out, se_v, buf, zbuf, mask_v):
    c = lax.axis_index("c")
    s = lax.axis_index("s")
    wid = s * NC + c
    b = wid // 2
    h = wid % 2

    pltpu.sync_copy(starts_hbm, se_v.at[pl.ds(0, 16)])
    pltpu.sync_copy(ends_hbm, se_v.at[pl.ds(16, 16)])
    lane = lax.broadcasted_iota(jnp.int32, (LANE,), 0)
    start = se_v[pl.ds(b, LANE)][0]
    end = se_v[pl.ds(16 + b, LANE)][0]
    seg = jnp.minimum(end - start, P)
    n_copy = jnp.clip(seg - h * HALF, 0, HALF)
    src0 = start + h * HALF
    dst0 = wid * HALF

    zero16 = jnp.zeros((LANE,), jnp.float32)

    def zrow(i, carry):
        base = i * D
        for k in range(D // LANE):
            zbuf[pl.ds(base + k * LANE, LANE)] = zero16
        return carry

    lax.fori_loop(0, CH, zrow, None)

    nfull = n_copy // CH
    rem = n_copy - nfull * CH

    def copy_chunk(i, carry):
        pltpu.sync_copy(flat.at[pl.ds((src0 + i * CH) * D, CH * D)], buf)
        pltpu.sync_copy(buf, out.at[pl.ds((dst0 + i * CH) * D, CH * D)])
        return carry

    lax.fori_loop(0, nfull, copy_chunk, None)

    base = nfull * CH
    for sz in (128, 64, 32, 16, 8, 4, 2, 1):
        off = base + (rem & ~(2 * sz - 1))

        @pl.when((rem & sz) != 0)
        def _(sz=sz, off=off):
            pltpu.sync_copy(
                flat.at[pl.ds((src0 + off) * D, sz * D)], buf.at[pl.ds(0, sz * D)]
            )
            pltpu.sync_copy(
                buf.at[pl.ds(0, sz * D)], out.at[pl.ds((dst0 + off) * D, sz * D)]
            )

    nz = HALF - n_copy
    zfull = nz // CH
    zrem = nz - zfull * CH
    for sz in (128, 64, 32, 16, 8, 4, 2, 1):
        off = n_copy + (zrem & ~(2 * sz - 1))

        @pl.when((zrem & sz) != 0)
        def _(sz=sz, off=off):
            pltpu.sync_copy(
                zbuf.at[pl.ds(0, sz * D)], out.at[pl.ds((dst0 + off) * D, sz * D)]
            )

    zbase = n_copy + zrem

    def zero_chunk(i, carry):
        pltpu.sync_copy(zbuf, out.at[pl.ds((dst0 + zbase + i * CH) * D, CH * D)])
        return carry

    lax.fori_loop(0, zfull, zero_chunk, None)

    def mrow(i, carry):
        j = i * LANE + lane
        mask_v[pl.ds(i * LANE, LANE)] = jnp.where(j < n_copy, 1.0, 0.0).astype(
            jnp.float32
        )
        return carry

    lax.fori_loop(0, HALF // LANE, mrow, None)
    pltpu.sync_copy(mask_v, mask_out.at[pl.ds(dst0, HALF)])


_padder = functools.partial(
    pl.kernel,
    mesh=_mesh,
    out_type=[
        jax.ShapeDtypeStruct((B * P * D,), jnp.float32),
        jax.ShapeDtypeStruct((B * P,), jnp.float32),
    ],
    scratch_types=[
        pltpu.VMEM((48,), jnp.int32),
        pltpu.VMEM((CH * D,), jnp.float32),
        pltpu.VMEM((CH * D,), jnp.float32),
        pltpu.VMEM((HALF,), jnp.float32),
    ],
)(_body)


@jax.jit
def kernel(flat, cu_seqlens):
    starts = cu_seqlens[:16]
    ends = cu_seqlens[1:17]
    padded_flat, mask_flat = _padder(flat.reshape(T * D), starts, ends)
    return padded_flat.reshape(B, P, D), mask_flat.reshape(B, P)


# speedup vs baseline: 1.6790x; 1.0311x over previous
import functools

import jax
import jax.numpy as jnp
from jax import lax
from jax.experimental import pallas as pl
from jax.experimental.pallas import tpu as pltpu
from jax.experimental.pallas import tpu_sc as plsc

B = 16
P = 4096
D = 192
T = 32768
NC = 2
NS = 16
NW = NC * NS
HALF = P // 2
CH = 128
NBUF = 3
LANE = 16

_mesh = plsc.VectorSubcoreMesh(core_axis_name="c", subcore_axis_name="s")


def _body(flat, starts_hbm, ends_hbm, out, mask_out, se_v, buf, zbuf, mask_v,
          sem_in0, sem_in1, sem_in2, sem_out0, sem_out1, sem_out2,
          sem_zero, sem_mask):
    sems_in = (sem_in0, sem_in1, sem_in2)
    sems_out = (sem_out0, sem_out1, sem_out2)
    c = lax.axis_index("c")
    s = lax.axis_index("s")
    wid = s * NC + c
    b = wid // 2
    h = wid % 2

    pltpu.sync_copy(starts_hbm, se_v.at[pl.ds(0, 16)])
    pltpu.sync_copy(ends_hbm, se_v.at[pl.ds(16, 16)])
    lane = lax.broadcasted_iota(jnp.int32, (LANE,), 0)
    start = se_v[pl.ds(b, LANE)][0]
    end = se_v[pl.ds(16 + b, LANE)][0]
    seg = jnp.minimum(end - start, P)
    n_copy = jnp.clip(seg - h * HALF, 0, HALF)
    src0 = start + h * HALF
    dst0 = wid * HALF

    nfull = n_copy // CH
    rem = n_copy - nfull * CH
    nlive = jnp.minimum(nfull, NBUF)

    def bslice(k):
        return buf.at[pl.ds(k * CH * D, CH * D)]

    def in_copy(i, k):
        pltpu.async_copy(
            flat.at[pl.ds((src0 + i * CH) * D, CH * D)], bslice(k), sems_in[k]
        )

    def out_copy(i, k):
        pltpu.async_copy(
            bslice(k), out.at[pl.ds((dst0 + i * CH) * D, CH * D)], sems_out[k]
        )

    def wait_in(k):
        pltpu.make_async_copy(
            flat.at[pl.ds(0, CH * D)], bslice(k), sems_in[k]
        ).wait()

    def wait_out(k):
        pltpu.make_async_copy(
            bslice(k), out.at[pl.ds(dst0 * D, CH * D)], sems_out[k]
        ).wait()

    for k in range(NBUF):
        @pl.when(k < nlive)
        def _(k=k):
            in_copy(k, k)

    zero16 = jnp.zeros((LANE,), jnp.float32)

    def zrow(i, carry):
        bs = i * D
        for kk in range(D // LANE):
            zbuf[pl.ds(bs + kk * LANE, LANE)] = zero16
        return carry

    lax.fori_loop(0, CH, zrow, None)

    nz = HALF - n_copy
    zfull = nz // CH
    zrem = nz - zfull * CH
    for sz in (64, 32, 16, 8, 4, 2, 1):
        off = n_copy + (zrem & ~(2 * sz - 1))

        @pl.when((zrem & sz) != 0)
        def _(sz=sz, off=off):
            pltpu.async_copy(
                zbuf.at[pl.ds(0, sz * D)],
                out.at[pl.ds((dst0 + off) * D, sz * D)],
                sem_zero,
            )

    zbase = n_copy + zrem

    def zero_chunk(i, carry):
        pltpu.async_copy(
            zbuf, out.at[pl.ds((dst0 + zbase + i * CH) * D, CH * D)], sem_zero
        )
        return carry

    lax.fori_loop(0, zfull, zero_chunk, None)

    def mrow(i, carry):
        j = i * LANE + lane
        mask_v[pl.ds(i * LANE, LANE)] = jnp.where(j < n_copy, 1.0, 0.0).astype(
            jnp.float32
        )
        return carry

    lax.fori_loop(0, HALF // LANE, mrow, None)
    pltpu.async_copy(mask_v, mask_out.at[pl.ds(dst0, HALF)], sem_mask)

    nrounds = (nfull + NBUF - 1) // NBUF

    def round_body(j, carry):
        for k in range(NBUF):
            i = j * NBUF + k

            @pl.when(i < nfull)
            def _(i=i, k=k):
                wait_in(k)
                out_copy(i, k)
        for k in range(NBUF):
            i = (j + 1) * NBUF + k

            @pl.when(i < nfull)
            def _(i=i, k=k):
                wait_out(k)
                in_copy(i, k)
        return carry

    lax.fori_loop(0, nrounds, round_body, None)
    for k in range(NBUF):
        @pl.when(k < nlive)
        def _(k=k):
            wait_out(k)

    base = nfull * CH
    for sz in (64, 32, 16, 8, 4, 2, 1):
        off = base + (rem & ~(2 * sz - 1))

        @pl.when((rem & sz) != 0)
        def _(sz=sz, off=off):
            pltpu.sync_copy(
                flat.at[pl.ds((src0 + off) * D, sz * D)],
                buf.at[pl.ds(0, sz * D)],
            )
            pltpu.sync_copy(
                buf.at[pl.ds(0, sz * D)],
                out.at[pl.ds((dst0 + off) * D, sz * D)],
            )

    for sz in (64, 32, 16, 8, 4, 2, 1):
        @pl.when((zrem & sz) != 0)
        def _(sz=sz):
            pltpu.make_async_copy(
                zbuf.at[pl.ds(0, sz * D)],
                out.at[pl.ds(dst0 * D, sz * D)],
                sem_zero,
            ).wait()

    def zero_drain(i, carry):
        pltpu.make_async_copy(
            zbuf, out.at[pl.ds(dst0 * D, CH * D)], sem_zero
        ).wait()
        return carry

    lax.fori_loop(0, zfull, zero_drain, None)
    pltpu.make_async_copy(mask_v, mask_out.at[pl.ds(dst0, HALF)], sem_mask).wait()


_padder = functools.partial(
    pl.kernel,
    mesh=_mesh,
    out_type=[
        jax.ShapeDtypeStruct((B * P * D,), jnp.float32),
        jax.ShapeDtypeStruct((B * P,), jnp.float32),
    ],
    scratch_types=[
        pltpu.VMEM((48,), jnp.int32),
        pltpu.VMEM((NBUF * CH * D,), jnp.float32),
        pltpu.VMEM((CH * D,), jnp.float32),
        pltpu.VMEM((HALF,), jnp.float32),
    ] + [pltpu.SemaphoreType.DMA] * 8,
)(_body)


@jax.jit
def kernel(flat, cu_seqlens):
    starts = cu_seqlens[:16]
    ends = cu_seqlens[1:17]
    padded_flat, mask_flat = _padder(flat.reshape(T * D), starts, ends)
    return padded_flat.reshape(B, P, D), mask_flat.reshape(B, P)


# speedup vs baseline: 1.6845x; 1.0033x over previous
import functools

import jax
import jax.numpy as jnp
from jax import lax
from jax.experimental import pallas as pl
from jax.experimental.pallas import tpu as pltpu
from jax.experimental.pallas import tpu_sc as plsc

B = 16
P = 4096
D = 192
T = 32768
NC = 2
NS = 16
NW = NC * NS
HALF = P // 2
CH = 128
NBUF = 3
LANE = 16

_mesh = plsc.VectorSubcoreMesh(core_axis_name="c", subcore_axis_name="s")


def _body(flat, starts_hbm, ends_hbm, out, mask_out, se_v, buf, zbuf, mask_v,
          sem_in0, sem_in1, sem_in2, sem_out0, sem_out1, sem_out2,
          sem_zero, sem_mask):
    sems_in = (sem_in0, sem_in1, sem_in2)
    sems_out = (sem_out0, sem_out1, sem_out2)
    c = lax.axis_index("c")
    s = lax.axis_index("s")
    wid = s * NC + c
    b = wid // 2
    h = wid % 2

    pltpu.sync_copy(starts_hbm, se_v.at[pl.ds(0, 16)])
    pltpu.sync_copy(ends_hbm, se_v.at[pl.ds(16, 16)])
    lane = lax.broadcasted_iota(jnp.int32, (LANE,), 0)
    start = se_v[pl.ds(b, LANE)][0]
    end = se_v[pl.ds(16 + b, LANE)][0]
    seg = jnp.minimum(end - start, P)
    n_copy = jnp.clip(seg - h * HALF, 0, HALF)
    src0 = start + h * HALF
    dst0 = wid * HALF

    nfull = n_copy // CH
    rem = n_copy - nfull * CH
    nlive = jnp.minimum(nfull, NBUF)

    def bslice(k):
        return buf.at[pl.ds(k * CH, CH)]

    def in_copy(i, k):
        pltpu.async_copy(
            flat.at[pl.ds(src0 + i * CH, CH)], bslice(k), sems_in[k]
        )

    def out_copy(i, k):
        pltpu.async_copy(
            bslice(k), out.at[pl.ds(dst0 + i * CH, CH)], sems_out[k]
        )

    def wait_in(k):
        pltpu.make_async_copy(
            flat.at[pl.ds(0, CH)], bslice(k), sems_in[k]
        ).wait()

    def wait_out(k):
        pltpu.make_async_copy(
            buf.at[pl.ds(0, CH)], out.at[pl.ds(dst0, CH)], sems_out[k]
        ).wait()

    for k in range(NBUF):
        @pl.when(k < nlive)
        def _(k=k):
            in_copy(k, k)

    zero16 = jnp.zeros((LANE,), jnp.float32)

    def zrow(i, carry):
        for kk in range(D // LANE):
            zbuf[i, pl.ds(kk * LANE, LANE)] = zero16
        return carry

    lax.fori_loop(0, CH, zrow, None)

    nz = HALF - n_copy
    zfull = nz // CH
    zrem = nz - zfull * CH
    for sz in (64, 32, 16, 8, 4, 2, 1):
        off = n_copy + (zrem & ~(2 * sz - 1))

        @pl.when((zrem & sz) != 0)
        def _(sz=sz, off=off):
            pltpu.async_copy(
                zbuf.at[pl.ds(0, sz)],
                out.at[pl.ds(dst0 + off, sz)],
                sem_zero,
            )

    zbase = n_copy + zrem

    def zero_chunk(i, carry):
        pltpu.async_copy(
            zbuf, out.at[pl.ds(dst0 + zbase + i * CH, CH)], sem_zero
        )
        return carry

    lax.fori_loop(0, zfull, zero_chunk, None)

    def mrow(i, carry):
        j = i * LANE + lane
        mask_v[pl.ds(i * LANE, LANE)] = jnp.where(j < n_copy, 1.0, 0.0).astype(
            jnp.float32
        )
        return carry

    lax.fori_loop(0, HALF // LANE, mrow, None)
    pltpu.async_copy(mask_v, mask_out.at[pl.ds(dst0, HALF)], sem_mask)

    nrounds = (nfull + NBUF - 1) // NBUF

    def round_body(j, carry):
        for k in range(NBUF):
            i = j * NBUF + k

            @pl.when(i < nfull)
            def _(i=i, k=k):
                wait_in(k)
                out_copy(i, k)
        for k in range(NBUF):
            i = (j + 1) * NBUF + k

            @pl.when(i < nfull)
            def _(i=i, k=k):
                wait_out(k)
                in_copy(i, k)
        return carry

    lax.fori_loop(0, nrounds, round_body, None)
    for k in range(NBUF):
        @pl.when(k < nlive)
        def _(k=k):
            wait_out(k)

    base = nfull * CH
    for sz in (64, 32, 16, 8, 4, 2, 1):
        off = base + (rem & ~(2 * sz - 1))

        @pl.when((rem & sz) != 0)
        def _(sz=sz, off=off):
            pltpu.sync_copy(
                flat.at[pl.ds(src0 + off, sz)], buf.at[pl.ds(0, sz)]
            )
            pltpu.sync_copy(
                buf.at[pl.ds(0, sz)], out.at[pl.ds(dst0 + off, sz)]
            )

    for sz in (64, 32, 16, 8, 4, 2, 1):
        @pl.when((zrem & sz) != 0)
        def _(sz=sz):
            pltpu.make_async_copy(
                zbuf.at[pl.ds(0, sz)], out.at[pl.ds(dst0, sz)], sem_zero
            ).wait()

    def zero_drain(i, carry):
        pltpu.make_async_copy(
            zbuf, out.at[pl.ds(dst0, CH)], sem_zero
        ).wait()
        return carry

    lax.fori_loop(0, zfull, zero_drain, None)
    pltpu.make_async_copy(mask_v, mask_out.at[pl.ds(dst0, HALF)], sem_mask).wait()


_padder = functools.partial(
    pl.kernel,
    mesh=_mesh,
    out_type=[
        jax.ShapeDtypeStruct((B * P, D), jnp.float32),
        jax.ShapeDtypeStruct((B * P,), jnp.float32),
    ],
    compiler_params=pltpu.CompilerParams(use_tc_tiling_on_sc=False),
    scratch_types=[
        pltpu.VMEM((48,), jnp.int32),
        pltpu.VMEM((NBUF * CH, D), jnp.float32),
        pltpu.VMEM((CH, D), jnp.float32),
        pltpu.VMEM((HALF,), jnp.float32),
    ] + [pltpu.SemaphoreType.DMA] * 8,
)(_body)


@jax.jit
def kernel(flat, cu_seqlens):
    starts = cu_seqlens[:16]
    ends = cu_seqlens[1:17]
    padded, mask_flat = _padder(flat, starts, ends)
    return padded.reshape(B, P, D), mask_flat.reshape(B, P)


# speedup vs baseline: 1.8013x; 1.0693x over previous
import functools

import jax
import jax.numpy as jnp
from jax import lax
from jax.experimental import pallas as pl
from jax.experimental.pallas import tpu as pltpu
from jax.experimental.pallas import tpu_sc as plsc

B = 16
P = 4096
D = 192
T = 32768
NC = 2
NS = 16
NW = NC * NS
HALF = P // 2
CH = 96
CHH = CH + 8
NBUF = 3
LANE = 16
NG = D // LANE

_mesh = plsc.VectorSubcoreMesh(core_axis_name="c", subcore_axis_name="s")


def _align(x):
    return pl.multiple_of(x, 8)


def _body(flat, starts_hbm, ends_hbm, out, mask_out, se_v, buf, zbuf, vbuf,
          bgroup, mask_v, sem_in0, sem_in1, sem_in2, sem_out0, sem_out1,
          sem_out2, sem_zero, sem_mask, sem_bg):
    sems_in = (sem_in0, sem_in1, sem_in2)
    sems_out = (sem_out0, sem_out1, sem_out2)
    c = lax.axis_index("c")
    s = lax.axis_index("s")
    wid = s * NC + c
    b = wid // 2
    h = wid % 2

    pltpu.sync_copy(starts_hbm, se_v.at[pl.ds(0, 16)])
    pltpu.sync_copy(ends_hbm, se_v.at[pl.ds(16, 16)])
    lane = lax.broadcasted_iota(jnp.int32, (LANE,), 0)
    start = se_v[pl.ds(b, LANE)][0]
    end = se_v[pl.ds(16 + b, LANE)][0]
    seg = jnp.minimum(end - start, P)
    n_copy = jnp.clip(seg - h * HALF, 0, HALF)
    src0 = start + h * HALF
    dst0 = wid * HALF
    d = src0 & 7

    n8 = n_copy & ~7
    v = n_copy - n8
    nfull = n8 // CH
    rem8 = n8 - nfull * CH
    nlive = jnp.minimum(nfull, NBUF)

    def src_window(o, rows):
        astart = jnp.minimum((src0 + o) & ~7, T - rows)
        return _align(astart), src0 + o - astart

    def bslice(k, rows=CHH):
        return buf.at[pl.ds(k * CHH, rows)]

    def in_copy(i, k):
        astart, _ = src_window(i * CH, CHH)
        pltpu.async_copy(flat.at[pl.ds(astart, CHH)], bslice(k), sems_in[k])

    def shift_rows(base, dd, nrows):
        @pl.when(dd != 0)
        def _():
            def srow(r, carry):
                for kk in range(NG):
                    buf[base + r, pl.ds(kk * LANE, LANE)] = buf[
                        base + r + dd, pl.ds(kk * LANE, LANE)
                    ]
                return carry

            lax.fori_loop(0, nrows, srow, None)

    def out_copy(i, k):
        pltpu.async_copy(
            bslice(k, CH),
            out.at[pl.ds(_align(dst0 + i * CH), CH)],
            sems_out[k],
        )

    def wait_in(k):
        pltpu.make_async_copy(
            flat.at[pl.ds(0, CHH)], bslice(k), sems_in[k]
        ).wait()

    def wait_out(k):
        pltpu.make_async_copy(
            buf.at[pl.ds(0, CH)], out.at[pl.ds(_align(dst0), CH)], sems_out[k]
        ).wait()

    for k in range(NBUF):
        @pl.when(k < nlive)
        def _(k=k):
            in_copy(k, k)

    @pl.when(v > 0)
    def _():
        astart, _ = src_window(n8, 16)
        pltpu.async_copy(flat.at[pl.ds(astart, 16)], vbuf, sem_bg)

    zero16 = jnp.zeros((LANE,), jnp.float32)

    def zrow(i, carry):
        for kk in range(NG):
            zbuf[i, pl.ds(kk * LANE, LANE)] = zero16
        return carry

    lax.fori_loop(0, CH, zrow, None)
    for j in range(8):
        for kk in range(NG):
            bgroup[j, pl.ds(kk * LANE, LANE)] = zero16

    nz8 = jnp.maximum(HALF - n8 - 8, 0)
    zfull = nz8 // CH
    zrem = nz8 - zfull * CH
    zb0 = n8 + 8
    for sz in (64, 32, 16, 8):
        off = zb0 + (zrem & ~(2 * sz - 1))

        @pl.when((zrem & sz) != 0)
        def _(sz=sz, off=off):
            pltpu.async_copy(
                zbuf.at[pl.ds(0, sz)],
                out.at[pl.ds(_align(dst0 + off), sz)],
                sem_zero,
            )

    zbase = zb0 + zrem

    def zero_chunk(i, carry):
        pltpu.async_copy(
            zbuf, out.at[pl.ds(_align(dst0 + zbase + i * CH), CH)], sem_zero
        )
        return carry

    lax.fori_loop(0, zfull, zero_chunk, None)

    def mrow(i, carry):
        j = i * LANE + lane
        mask_v[pl.ds(i * LANE, LANE)] = jnp.where(j < n_copy, 1.0, 0.0).astype(
            jnp.float32
        )
        return carry

    lax.fori_loop(0, HALF // LANE, mrow, None)

    nrounds = (nfull + NBUF - 1) // NBUF

    def round_body(j, carry):
        for k in range(NBUF):
            i = j * NBUF + k

            @pl.when(i < nfull)
            def _(i=i, k=k):
                wait_in(k)
                _, dd = src_window(i * CH, CHH)
                shift_rows(k * CHH, dd, CH)
                out_copy(i, k)
        for k in range(NBUF):
            i = (j + 1) * NBUF + k

            @pl.when(i < nfull)
            def _(i=i, k=k):
                wait_out(k)
                in_copy(i, k)
        return carry

    lax.fori_loop(0, nrounds, round_body, None)

    for k in range(NBUF):
        @pl.when(k < nlive)
        def _(k=k):
            wait_out(k)

    base = nfull * CH
    for sz in (64, 32, 16, 8):
        off = base + (rem8 & ~(2 * sz - 1))

        @pl.when((rem8 & sz) != 0)
        def _(sz=sz, off=off):
            astart, dd = src_window(off, sz + 8)
            pltpu.sync_copy(
                flat.at[pl.ds(astart, sz + 8)], buf.at[pl.ds(0, sz + 8)]
            )
            shift_rows(0, dd, sz)
            pltpu.sync_copy(
                buf.at[pl.ds(0, sz)], out.at[pl.ds(_align(dst0 + off), sz)]
            )

    @pl.when(v > 0)
    def _():
        _, d_b = src_window(n8, 16)
        pltpu.make_async_copy(flat.at[pl.ds(0, 16)], vbuf, sem_bg).wait()
        for j in range(7):
            @pl.when(j < v)
            def _(j=j):
                for kk in range(NG):
                    bgroup[j, pl.ds(kk * LANE, LANE)] = vbuf[
                        d_b + j, pl.ds(kk * LANE, LANE)
                    ]

    @pl.when(n8 < HALF)
    def _():
        pltpu.async_copy(
            bgroup, out.at[pl.ds(_align(dst0 + n8), 8)], sem_zero
        )

    pltpu.async_copy(mask_v, mask_out.at[pl.ds(dst0, HALF)], sem_mask)

    for sz in (64, 32, 16, 8):
        @pl.when((zrem & sz) != 0)
        def _(sz=sz):
            pltpu.make_async_copy(
                zbuf.at[pl.ds(0, sz)],
                out.at[pl.ds(_align(dst0), sz)],
                sem_zero,
            ).wait()

    def zero_drain(i, carry):
        pltpu.make_async_copy(
            zbuf, out.at[pl.ds(_align(dst0), CH)], sem_zero
        ).wait()
        return carry

    lax.fori_loop(0, zfull, zero_drain, None)

    @pl.when(n8 < HALF)
    def _():
        pltpu.make_async_copy(
            bgroup, out.at[pl.ds(_align(dst0), 8)], sem_zero
        ).wait()

    pltpu.make_async_copy(mask_v, mask_out.at[pl.ds(dst0, HALF)], sem_mask).wait()


_padder = functools.partial(
    pl.kernel,
    mesh=_mesh,
    out_type=[
        jax.ShapeDtypeStruct((B * P, D), jnp.float32),
        jax.ShapeDtypeStruct((B * P,), jnp.float32),
    ],
    scratch_types=[
        pltpu.VMEM((48,), jnp.int32),
        pltpu.VMEM((NBUF * CHH, D), jnp.float32),
        pltpu.VMEM((CH, D), jnp.float32),
        pltpu.VMEM((16, D), jnp.float32),
        pltpu.VMEM((8, D), jnp.float32),
        pltpu.VMEM((HALF,), jnp.float32),
    ] + [pltpu.SemaphoreType.DMA] * 9,
)(_body)


@jax.jit
def kernel(flat, cu_seqlens):
    starts = cu_seqlens[:16]
    ends = cu_seqlens[1:17]
    padded, mask_flat = _padder(flat, starts, ends)
    return padded.reshape(B, P, D), mask_flat.reshape(B, P)


# speedup vs baseline: 2.6110x; 1.4495x over previous
import functools

import jax
import jax.numpy as jnp
from jax import lax
from jax.experimental import pallas as pl
from jax.experimental.pallas import tpu as pltpu
from jax.experimental.pallas import tpu_sc as plsc

B = 16
P = 4096
D = 192
T = 32768
NC = 2
NS = 16
NW = NC * NS
HALF = P // 2
CH = 64
CHH = CH + 8
NBUF = 3
LANE = 16
NG = D // LANE

_mesh = plsc.VectorSubcoreMesh(core_axis_name="c", subcore_axis_name="s")


def _align(x):
    return pl.multiple_of(x, 8)


def _body(flat, starts_hbm, ends_hbm, out, mask_out, se_v, bin_, bout, zbuf,
          vbuf, bgroup, mask_v, sem_in0, sem_in1, sem_in2, sem_out0,
          sem_out1, sem_out2, sem_zero, sem_mask, sem_bg):
    sems_in = (sem_in0, sem_in1, sem_in2)
    sems_out = (sem_out0, sem_out1, sem_out2)
    c = lax.axis_index("c")
    s = lax.axis_index("s")
    wid = s * NC + c
    b = wid // 2
    h = wid % 2

    pltpu.sync_copy(starts_hbm, se_v.at[pl.ds(0, 16)])
    pltpu.sync_copy(ends_hbm, se_v.at[pl.ds(16, 16)])
    lane = lax.broadcasted_iota(jnp.int32, (LANE,), 0)
    start = se_v[pl.ds(b, LANE)][0]
    end = se_v[pl.ds(16 + b, LANE)][0]
    seg = jnp.minimum(end - start, P)
    n_copy = jnp.clip(seg - h * HALF, 0, HALF)
    src0 = start + h * HALF
    dst0 = wid * HALF

    n8 = n_copy & ~7
    v = n_copy - n8
    nfull = n8 // CH
    rem8 = n8 - nfull * CH
    nlive = jnp.minimum(nfull, NBUF)

    def src_window(o, rows):
        astart = jnp.minimum((src0 + o) & ~7, T - rows)
        return _align(astart), src0 + o - astart

    def in_copy(i, k):
        astart, _ = src_window(i * CH, CHH)
        pltpu.async_copy(
            flat.at[pl.ds(astart, CHH)],
            bin_.at[pl.ds(k * CHH, CHH)],
            sems_in[k],
        )

    def out_copy(i, k):
        _, dd = src_window(i * CH, CHH)
        dst = out.at[pl.ds(_align(dst0 + i * CH), CH)]

        @pl.when(dd != 0)
        def _():
            @plsc.parallel_loop(0, CH, unroll=2)
            def _(r):
                for kk in range(NG):
                    bout[k * CH + r, pl.ds(kk * LANE, LANE)] = bin_[
                        k * CHH + r + dd, pl.ds(kk * LANE, LANE)
                    ]

            pltpu.async_copy(
                bout.at[pl.ds(k * CH, CH)], dst, sems_out[k]
            )

        @pl.when(dd == 0)
        def _():
            pltpu.async_copy(
                bin_.at[pl.ds(k * CHH, CH)], dst, sems_out[k]
            )

    def wait_in(k):
        pltpu.make_async_copy(
            flat.at[pl.ds(0, CHH)], bin_.at[pl.ds(k * CHH, CHH)], sems_in[k]
        ).wait()

    def wait_out(k):
        pltpu.make_async_copy(
            bout.at[pl.ds(0, CH)],
            out.at[pl.ds(_align(dst0), CH)],
            sems_out[k],
        ).wait()

    for k in range(NBUF):
        @pl.when(k < nlive)
        def _(k=k):
            in_copy(k, k)

    @pl.when(v > 0)
    def _():
        astart, _ = src_window(n8, 16)
        pltpu.async_copy(flat.at[pl.ds(astart, 16)], vbuf, sem_bg)

    zero16 = jnp.zeros((LANE,), jnp.float32)

    @plsc.parallel_loop(0, CH, unroll=2)
    def _(i):
        for kk in range(NG):
            zbuf[i, pl.ds(kk * LANE, LANE)] = zero16

    for j in range(8):
        for kk in range(NG):
            bgroup[j, pl.ds(kk * LANE, LANE)] = zero16

    nz8 = jnp.maximum(HALF - n8 - 8, 0)
    zfull = nz8 // CH
    zrem = nz8 - zfull * CH
    zb0 = n8 + 8
    for sz in (32, 16, 8):
        off = zb0 + (zrem & ~(2 * sz - 1))

        @pl.when((zrem & sz) != 0)
        def _(sz=sz, off=off):
            pltpu.async_copy(
                zbuf.at[pl.ds(0, sz)],
                out.at[pl.ds(_align(dst0 + off), sz)],
                sem_zero,
            )

    zbase = zb0 + zrem

    def zero_chunk(i, carry):
        pltpu.async_copy(
            zbuf, out.at[pl.ds(_align(dst0 + zbase + i * CH), CH)], sem_zero
        )
        return carry

    lax.fori_loop(0, zfull, zero_chunk, None)

    @plsc.parallel_loop(0, HALF // LANE, unroll=2)
    def _(i):
        j = i * LANE + lane
        mask_v[pl.ds(i * LANE, LANE)] = jnp.where(j < n_copy, 1.0, 0.0).astype(
            jnp.float32
        )

    nrounds = (nfull + NBUF - 1) // NBUF

    def round_body(j, carry):
        for k in range(NBUF):
            i = j * NBUF + k

            @pl.when(i < nfull)
            def _(i=i, k=k):
                wait_in(k)
                out_copy(i, k)
        for k in range(NBUF):
            i = (j + 1) * NBUF + k

            @pl.when(i < nfull)
            def _(i=i, k=k):
                wait_out(k)
                in_copy(i, k)
        return carry

    lax.fori_loop(0, nrounds, round_body, None)

    for k in range(NBUF):
        @pl.when(k < nlive)
        def _(k=k):
            wait_out(k)

    base = nfull * CH
    for sz in (32, 16, 8):
        off = base + (rem8 & ~(2 * sz - 1))

        @pl.when((rem8 & sz) != 0)
        def _(sz=sz, off=off):
            astart, dd = src_window(off, sz + 8)
            pltpu.sync_copy(
                flat.at[pl.ds(astart, sz + 8)], bin_.at[pl.ds(0, sz + 8)]
            )

            @pl.when(dd != 0)
            def _():
                def srow(r, carry):
                    for kk in range(NG):
                        bin_[r, pl.ds(kk * LANE, LANE)] = bin_[
                            r + dd, pl.ds(kk * LANE, LANE)
                        ]
                    return carry

                lax.fori_loop(0, sz, srow, None)

            pltpu.sync_copy(
                bin_.at[pl.ds(0, sz)], out.at[pl.ds(_align(dst0 + off), sz)]
            )

    @pl.when(v > 0)
    def _():
        _, d_b = src_window(n8, 16)
        pltpu.make_async_copy(flat.at[pl.ds(0, 16)], vbuf, sem_bg).wait()
        for j in range(7):
            @pl.when(j < v)
            def _(j=j):
                for kk in range(NG):
                    bgroup[j, pl.ds(kk * LANE, LANE)] = vbuf[
                        d_b + j, pl.ds(kk * LANE, LANE)
                    ]

    @pl.when(n8 < HALF)
    def _():
        pltpu.async_copy(
            bgroup, out.at[pl.ds(_align(dst0 + n8), 8)], sem_zero
        )

    pltpu.async_copy(mask_v, mask_out.at[pl.ds(dst0, HALF)], sem_mask)

    for sz in (32, 16, 8):
        @pl.when((zrem & sz) != 0)
        def _(sz=sz):
            pltpu.make_async_copy(
                zbuf.at[pl.ds(0, sz)],
                out.at[pl.ds(_align(dst0), sz)],
                sem_zero,
            ).wait()

    def zero_drain(i, carry):
        pltpu.make_async_copy(
            zbuf, out.at[pl.ds(_align(dst0), CH)], sem_zero
        ).wait()
        return carry

    lax.fori_loop(0, zfull, zero_drain, None)

    @pl.when(n8 < HALF)
    def _():
        pltpu.make_async_copy(
            bgroup, out.at[pl.ds(_align(dst0), 8)], sem_zero
        ).wait()

    pltpu.make_async_copy(mask_v, mask_out.at[pl.ds(dst0, HALF)], sem_mask).wait()


_padder = functools.partial(
    pl.kernel,
    mesh=_mesh,
    out_type=[
        jax.ShapeDtypeStruct((B * P, D), jnp.float32),
        jax.ShapeDtypeStruct((B * P,), jnp.float32),
    ],
    scratch_types=[
        pltpu.VMEM((48,), jnp.int32),
        pltpu.VMEM((NBUF * CHH, D), jnp.float32),
        pltpu.VMEM((NBUF * CH, D), jnp.float32),
        pltpu.VMEM((CH, D), jnp.float32),
        pltpu.VMEM((16, D), jnp.float32),
        pltpu.VMEM((8, D), jnp.float32),
        pltpu.VMEM((HALF,), jnp.float32),
    ] + [pltpu.SemaphoreType.DMA] * 9,
)(_body)


@jax.jit
def kernel(flat, cu_seqlens):
    starts = cu_seqlens[:16]
    ends = cu_seqlens[1:17]
    padded, mask_flat = _padder(flat, starts, ends)
    return padded.reshape(B, P, D), mask_flat.reshape(B, P)
